# Initial kernel scaffold; baseline (speedup 1.0000x reference)
import jax
import jax.numpy as jnp
from jax.experimental import pallas as pl


def kernel(x, edge_index, edge_attr, batch, W_l, W_r, att, W_e, b_gat, W_lin, b_lin):
    raise NotImplementedError("write your pallas kernel here")


# speedup vs baseline: 1.8553x; 1.8553x over previous
import functools

import jax
import jax.numpy as jnp
from jax import lax
from jax.experimental import pallas as pl
from jax.experimental.pallas import tpu as pltpu
from jax.experimental.pallas import tpu_sc as plsc

NC = 2
NS = 16
L = 16

F_OUT = 128
CHUNK = 200
CHUNK_PAD = 208
NEG_SLOPE = 0.2


def _mm_body(x_ref, wl_ref, wr_ref, xl_ref, xr_ref):
    xb = x_ref[...]
    xl_ref[...] = jnp.dot(xb, wl_ref[...], preferred_element_type=jnp.float32)
    xr_ref[...] = jnp.dot(xb, wr_ref[...], preferred_element_type=jnp.float32)


def _project(x, W_l, W_r):
    n, f_in = x.shape
    bm = 1000
    grid = n // bm
    return pl.pallas_call(
        _mm_body,
        grid=(grid,),
        in_specs=[
            pl.BlockSpec((bm, f_in), lambda i: (i, 0)),
            pl.BlockSpec((f_in, F_OUT), lambda i: (0, 0)),
            pl.BlockSpec((f_in, F_OUT), lambda i: (0, 0)),
        ],
        out_specs=[
            pl.BlockSpec((bm, F_OUT), lambda i: (i, 0)),
            pl.BlockSpec((bm, F_OUT), lambda i: (i, 0)),
        ],
        out_shape=[
            jax.ShapeDtypeStruct((n, F_OUT), jnp.float32),
            jax.ShapeDtypeStruct((n, F_OUT), jnp.float32),
        ],
    )(x, W_l, W_r)


def _logits_body(xl_hbm, xr_hbm, src_hbm, dst_hbm, ea_hbm, attwe_hbm,
                 lg_hbm, wmax_hbm,
                 src_v, dst_v, ea_v, attwe_v, lg_v, wmax_v,
                 xl_rows, xr_rows, sem):
    c = lax.axis_index("c")
    s = lax.axis_index("s")
    wid = c * NS + s
    e_total = lg_hbm.shape[0]
    per_w = e_total // (NC * NS)
    n_chunks = per_w // CHUNK
    base_w = wid * per_w

    pltpu.sync_copy(attwe_hbm, attwe_v)
    lanes = lax.iota(jnp.int32, L)

    def chunk_body(ci, wmax):
        base = base_w + ci * CHUNK
        pltpu.sync_copy(src_hbm.at[pl.ds(base, CHUNK)], src_v)
        pltpu.sync_copy(dst_hbm.at[pl.ds(base, CHUNK)], dst_v)
        pltpu.sync_copy(ea_hbm.at[pl.ds(base, CHUNK)], ea_v)
        pltpu.async_copy(xl_hbm.at[src_v], xl_rows, sem).wait()
        pltpu.async_copy(xr_hbm.at[dst_v], xr_rows, sem).wait()

        def group_body(g, wmax_in):
            e0 = g * L
            eidx = jnp.minimum(e0 + lanes, CHUNK - 1)
            ea_g = plsc.load_gather(ea_v, [eidx])

            def feat_body(j, acc):
                jsplat = jnp.full((L,), j, jnp.int32)
                xlv = plsc.load_gather(xl_rows, [eidx, jsplat])
                xrv = plsc.load_gather(xr_rows, [eidx, jsplat])
                att_j = attwe_v[j]
                we_j = attwe_v[j + F_OUT]
                m = xlv + xrv + ea_g * we_j
                m = jnp.maximum(m, m * NEG_SLOPE)
                return acc + m * att_j

            acc = lax.fori_loop(0, F_OUT, feat_body, jnp.zeros((L,), jnp.float32),
                                unroll=8)
            lg_v[pl.ds(e0, L)] = acc
            return jnp.maximum(wmax_in, acc)

        n_groups = (CHUNK + L - 1) // L
        wmax = lax.fori_loop(0, n_groups, group_body, wmax)
        pltpu.sync_copy(lg_v.at[pl.ds(0, CHUNK)], lg_hbm.at[pl.ds(base, CHUNK)])
        return wmax

    wmax0 = jnp.full((L,), -jnp.inf, jnp.float32)
    wmax = lax.fori_loop(0, n_chunks, chunk_body, wmax0)
    wmax_v[...] = wmax
    pltpu.sync_copy(wmax_v, wmax_hbm.at[pl.ds(wid * L, L)])


def _aggregate_body(xl_hbm, src_hbm, dst_hbm, lg_hbm, wmax_hbm,
                    agg_hbm, den_hbm,
                    src_v, dst_v, lg_v, ex_v, wmax_v, xl_rows, zbuf,
                    agg_sp, den_sp, sem):
    c = lax.axis_index("c")
    s = lax.axis_index("s")
    wid = c * NS + s
    e_total = lg_hbm.shape[0]
    n = xl_hbm.shape[0]
    per_w = e_total // (NC * NS)
    n_chunks = per_w // CHUNK
    base_w = wid * per_w

    n_t0 = (n // NS) // 8 * 8
    n_last = n - n_t0 * (NS - 1)
    nd_pad = den_sp.shape[0]
    nd_tile = nd_pad // NS

    zv = jnp.zeros((L,), jnp.float32)

    def zrow(r, cr):
        def zcol(k, ck):
            xl_rows[r, pl.ds(k * L, L)] = zv
            return ck
        return lax.fori_loop(0, F_OUT // L, zcol, cr)

    lax.fori_loop(0, CHUNK, zrow, jnp.int32(0))

    def zflat(k, ck):
        zbuf[pl.ds(k * L, L)] = zv
        return ck

    lax.fori_loop(0, zbuf.shape[0] // L, zflat, jnp.int32(0))

    def _zero_agg_rows(r0, total):
        done = 0
        while total - done >= CHUNK:
            pltpu.sync_copy(xl_rows, agg_sp.at[pl.ds(r0 + done, CHUNK)])
            done += CHUNK
        rem = total - done
        if rem:
            pltpu.sync_copy(xl_rows.at[pl.ds(0, rem)],
                            agg_sp.at[pl.ds(r0 + done, rem)])

    @pl.when(s < NS - 1)
    def _():
        _zero_agg_rows(s * n_t0, n_t0)

    @pl.when(s == NS - 1)
    def _():
        _zero_agg_rows((NS - 1) * n_t0, n_last)

    pltpu.sync_copy(zbuf.at[pl.ds(0, nd_tile)],
                    den_sp.at[pl.ds(s * nd_tile, nd_tile)])

    pltpu.sync_copy(wmax_hbm, wmax_v)
    gvec = wmax_v[pl.ds(0, L)]
    for k in range(1, NC * NS):
        gvec = jnp.maximum(gvec, wmax_v[pl.ds(k * L, L)])
    gmax = jnp.max(gvec)

    plsc.subcore_barrier()

    lanes = lax.iota(jnp.int32, L)

    def chunk_body(ci, carry):
        base = base_w + ci * CHUNK
        pltpu.sync_copy(src_hbm.at[pl.ds(base, CHUNK)], src_v)
        pltpu.sync_copy(dst_hbm.at[pl.ds(base, CHUNK)], dst_v)
        pltpu.sync_copy(lg_hbm.at[pl.ds(base, CHUNK)], lg_v.at[pl.ds(0, CHUNK)])
        pltpu.async_copy(xl_hbm.at[src_v], xl_rows, sem).wait()

        def group_body(g, carry_in):
            e0 = g * L
            eidx = jnp.minimum(e0 + lanes, CHUNK - 1)
            lg = plsc.load_gather(lg_v, [eidx])
            ex = jnp.exp(lg - gmax)
            ex_v[pl.ds(e0, L)] = ex

            def feat_body(j, carry_f):
                jsplat = jnp.full((L,), j, jnp.int32)
                xlv = plsc.load_gather(xl_rows, [eidx, jsplat])
                plsc.store_scatter(xl_rows, [eidx, jsplat], ex * xlv)
                return carry_f

            return lax.fori_loop(0, F_OUT, feat_body, carry_in, unroll=8)

        n_groups = (CHUNK + L - 1) // L
        carry = lax.fori_loop(0, n_groups, group_body, carry)
        pltpu.sync_copy(ex_v.at[pl.ds(0, CHUNK)], den_sp.at[dst_v], add=True)
        pltpu.sync_copy(xl_rows, agg_sp.at[dst_v], add=True)
        return carry

    lax.fori_loop(0, n_chunks, chunk_body, jnp.int32(0))

    plsc.subcore_barrier()

    def _drain_agg_rows(r0, total):
        done = 0
        while done < total:
            sz = min(CHUNK, total - done)
            pltpu.sync_copy(agg_sp.at[pl.ds(r0 + done, sz)],
                            xl_rows.at[pl.ds(0, sz)])
            pltpu.sync_copy(xl_rows.at[pl.ds(0, sz)],
                            agg_hbm.at[c, pl.ds(r0 + done, sz)])
            done += sz

    @pl.when(s < NS - 1)
    def _():
        _drain_agg_rows(s * n_t0, n_t0)

    @pl.when(s == NS - 1)
    def _():
        _drain_agg_rows((NS - 1) * n_t0, n_last)

    pltpu.sync_copy(den_sp.at[pl.ds(s * nd_tile, nd_tile)],
                    zbuf.at[pl.ds(0, nd_tile)])
    pltpu.sync_copy(zbuf.at[pl.ds(0, nd_tile)],
                    den_hbm.at[pl.ds(c * nd_pad + s * nd_tile, nd_tile)])


def _finish_body(agg_ref, d0_ref, d1_ref, batch_ref, bgat_ref, wlin_ref,
                 blin_ref, out_ref, pooled_acc, cnt_acc):
    i = pl.program_id(0)
    nb = pl.num_programs(0)
    num_graphs = out_ref.shape[0]

    @pl.when(i == 0)
    def _():
        pooled_acc[...] = jnp.zeros_like(pooled_acc)
        cnt_acc[...] = jnp.zeros_like(cnt_acc)

    den = d0_ref[...] + d1_ref[...] + 1e-16
    h = (agg_ref[0] + agg_ref[1]) / den + bgat_ref[...]
    h = jnp.maximum(h, 0.0)

    gids = lax.broadcasted_iota(jnp.int32, (1, num_graphs), 1)
    oneh = (batch_ref[...] == gids).astype(jnp.float32)
    dims = (((0,), (0,)), ((), ()))
    pooled_acc[...] += lax.dot_general(oneh, h, dims,
                                       preferred_element_type=jnp.float32)
    ones = jnp.ones(h.shape, jnp.float32)
    cnt_acc[...] += lax.dot_general(oneh, ones, dims,
                                    preferred_element_type=jnp.float32)

    @pl.when(i == nb - 1)
    def _():
        pooled = pooled_acc[...] / jnp.maximum(cnt_acc[...], 1.0)
        out_ref[...] = jnp.dot(pooled, wlin_ref[...],
                               preferred_element_type=jnp.float32) + blin_ref[...]


def _finish(agg2, d0, d1, batch2, bgat2, W_lin, blin2, num_graphs, num_classes):
    n = agg2.shape[1]
    bm = 1000
    grid = n // bm
    return pl.pallas_call(
        _finish_body,
        grid=(grid,),
        in_specs=[
            pl.BlockSpec((NC, bm, F_OUT), lambda i: (0, i, 0)),
            pl.BlockSpec((bm, 1), lambda i: (i, 0)),
            pl.BlockSpec((bm, 1), lambda i: (i, 0)),
            pl.BlockSpec((bm, 1), lambda i: (i, 0)),
            pl.BlockSpec((1, F_OUT), lambda i: (0, 0)),
            pl.BlockSpec((F_OUT, num_classes), lambda i: (0, 0)),
            pl.BlockSpec((1, num_classes), lambda i: (0, 0)),
        ],
        out_specs=pl.BlockSpec((num_graphs, num_classes), lambda i: (0, 0)),
        out_shape=jax.ShapeDtypeStruct((num_graphs, num_classes), jnp.float32),
        scratch_shapes=[
            pltpu.VMEM((num_graphs, F_OUT), jnp.float32),
            pltpu.VMEM((num_graphs, F_OUT), jnp.float32),
        ],
    )(agg2, d0, d1, batch2, bgat2, W_lin, blin2)


def kernel(x, edge_index, edge_attr, batch, W_l, W_r, att, W_e, b_gat,
           W_lin, b_lin):
    n = x.shape[0]
    e = edge_index.shape[1]
    num_graphs = 64
    num_classes = W_lin.shape[1]
    nd_pad = ((n + NS * 8 - 1) // (NS * 8)) * (NS * 8)

    x_l, x_r = _project(x, W_l, W_r)

    src = edge_index[0]
    dst = edge_index[1]
    ea = edge_attr[:, 0]
    attwe = jnp.concatenate([att, W_e[0]])[:, None] * jnp.ones((1, L), jnp.float32)

    mesh = plsc.VectorSubcoreMesh(core_axis_name="c", subcore_axis_name="s",
                                  num_cores=NC, num_subcores=NS)

    logits_fn = pl.kernel(
        _logits_body,
        out_type=[
            jax.ShapeDtypeStruct((e,), jnp.float32),
            jax.ShapeDtypeStruct((NC * NS * L,), jnp.float32),
        ],
        mesh=mesh,
        compiler_params=pltpu.CompilerParams(needs_layout_passes=False),
        scratch_types=[
            pltpu.VMEM((CHUNK,), jnp.int32),
            pltpu.VMEM((CHUNK,), jnp.int32),
            pltpu.VMEM((CHUNK,), jnp.float32),
            pltpu.VMEM((2 * F_OUT, L), jnp.float32),
            pltpu.VMEM((CHUNK_PAD,), jnp.float32),
            pltpu.VMEM((L,), jnp.float32),
            pltpu.VMEM((CHUNK, F_OUT), jnp.float32),
            pltpu.VMEM((CHUNK, F_OUT), jnp.float32),
            pltpu.SemaphoreType.DMA,
        ],
    )
    logits, wmax = logits_fn(x_l, x_r, src, dst, ea, attwe)

    agg_fn = pl.kernel(
        _aggregate_body,
        out_type=[
            jax.ShapeDtypeStruct((NC, n, F_OUT), jnp.float32),
            jax.ShapeDtypeStruct((NC * nd_pad,), jnp.float32),
        ],
        mesh=mesh,
        compiler_params=pltpu.CompilerParams(needs_layout_passes=False),
        scratch_types=[
            pltpu.VMEM((CHUNK,), jnp.int32),
            pltpu.VMEM((CHUNK,), jnp.int32),
            pltpu.VMEM((CHUNK_PAD,), jnp.float32),
            pltpu.VMEM((CHUNK_PAD,), jnp.float32),
            pltpu.VMEM((NC * NS * L,), jnp.float32),
            pltpu.VMEM((CHUNK, F_OUT), jnp.float32),
            pltpu.VMEM((640,), jnp.float32),
            pltpu.VMEM_SHARED((n, F_OUT), jnp.float32),
            pltpu.VMEM_SHARED((nd_pad,), jnp.float32),
            pltpu.SemaphoreType.DMA,
        ],
    )
    agg2, den2 = agg_fn(x_l, src, dst, logits, wmax)

    d0 = den2[:n, None]
    d1 = den2[nd_pad:nd_pad + n, None]
    batch2 = batch[:, None]
    bgat2 = b_gat[None, :]
    blin2 = b_lin[None, :]
    return _finish(agg2, d0, d1, batch2, bgat2, W_lin, blin2,
                   num_graphs, num_classes)


# speedup vs baseline: 5.6768x; 3.0597x over previous
import jax
import jax.numpy as jnp
from jax import lax
from jax.experimental import pallas as pl
from jax.experimental.pallas import tpu as pltpu
from jax.experimental.pallas import tpu_sc as plsc

NC = 2
NS = 16
L = 16

F_OUT = 128
CHUNK = 400
CHUNK_S = 200
BM_E = 2000
NEG_SLOPE = 0.2


def _mm_body(x_ref, wl_ref, wr_ref, xl_ref, xr_ref):
    xb = x_ref[...]
    xl_ref[...] = jnp.dot(xb, wl_ref[...], preferred_element_type=jnp.float32)
    xr_ref[...] = jnp.dot(xb, wr_ref[...], preferred_element_type=jnp.float32)


def _project(x, W_l, W_r):
    n, f_in = x.shape
    bm = 1000
    return pl.pallas_call(
        _mm_body,
        grid=(n // bm,),
        in_specs=[
            pl.BlockSpec((bm, f_in), lambda i: (i, 0)),
            pl.BlockSpec((f_in, F_OUT), lambda i: (0, 0)),
            pl.BlockSpec((f_in, F_OUT), lambda i: (0, 0)),
        ],
        out_specs=[
            pl.BlockSpec((bm, F_OUT), lambda i: (i, 0)),
            pl.BlockSpec((bm, F_OUT), lambda i: (i, 0)),
        ],
        out_shape=[
            jax.ShapeDtypeStruct((n, F_OUT), jnp.float32),
            jax.ShapeDtypeStruct((n, F_OUT), jnp.float32),
        ],
    )(x, W_l, W_r)


def _logits_body(xlr_ref, xrr_ref, ea_ref, we_ref, att_ref,
                 lg_ref, bmax_ref, max_acc):
    i = pl.program_id(0)
    nb = pl.num_programs(0)
    m = xlr_ref[...] + xrr_ref[...] + ea_ref[...] * we_ref[...]
    m = jnp.maximum(m, m * NEG_SLOPE)
    lg = jnp.dot(m, att_ref[...], preferred_element_type=jnp.float32)
    lg_ref[...] = lg

    @pl.when(i == 0)
    def _():
        max_acc[...] = jnp.full_like(max_acc, -jnp.inf)

    max_acc[...] = jnp.maximum(max_acc[...], jnp.max(lg))

    @pl.when(i == nb - 1)
    def _():
        bmax_ref[...] = max_acc[...]


def _edge_logits(xlr, xrr, ea2, we2, att2):
    e = xlr.shape[0]
    nb = e // BM_E
    return pl.pallas_call(
        _logits_body,
        grid=(nb,),
        in_specs=[
            pl.BlockSpec((BM_E, F_OUT), lambda i: (i, 0)),
            pl.BlockSpec((BM_E, F_OUT), lambda i: (i, 0)),
            pl.BlockSpec((BM_E, 1), lambda i: (i, 0)),
            pl.BlockSpec((1, F_OUT), lambda i: (0, 0)),
            pl.BlockSpec((F_OUT, 1), lambda i: (0, 0)),
        ],
        out_specs=[
            pl.BlockSpec((BM_E, 1), lambda i: (i, 0)),
            pl.BlockSpec((1, F_OUT), lambda i: (0, 0)),
        ],
        out_shape=[
            jax.ShapeDtypeStruct((e, 1), jnp.float32),
            jax.ShapeDtypeStruct((1, F_OUT), jnp.float32),
        ],
        scratch_shapes=[pltpu.VMEM((1, F_OUT), jnp.float32)],
    )(xlr, xrr, ea2, we2, att2)


def _scale_body(lg_ref, xlr_ref, bmax_ref, ex_ref, scaled_ref):
    gmax = jnp.max(bmax_ref[...])
    ex = jnp.exp(lg_ref[...] - gmax)
    ex_ref[...] = ex
    scaled_ref[...] = xlr_ref[...] * ex


def _edge_scale(lg, xlr, bmax):
    e = xlr.shape[0]
    nb = e // BM_E
    return pl.pallas_call(
        _scale_body,
        grid=(nb,),
        in_specs=[
            pl.BlockSpec((BM_E, 1), lambda i: (i, 0)),
            pl.BlockSpec((BM_E, F_OUT), lambda i: (i, 0)),
            pl.BlockSpec((1, F_OUT), lambda i: (0, 0)),
        ],
        out_specs=[
            pl.BlockSpec((BM_E, 1), lambda i: (i, 0)),
            pl.BlockSpec((BM_E, F_OUT), lambda i: (i, 0)),
        ],
        out_shape=[
            jax.ShapeDtypeStruct((e, 1), jnp.float32),
            jax.ShapeDtypeStruct((e, F_OUT), jnp.float32),
        ],
    )(lg, xlr, bmax)


def _finish_body(agg_ref, d0_ref, d1_ref, batch_ref, bgat_ref, wlin_ref,
                 blin_ref, out_ref, pooled_acc, cnt_acc):
    i = pl.program_id(0)
    nb = pl.num_programs(0)
    num_graphs = out_ref.shape[0]

    @pl.when(i == 0)
    def _():
        pooled_acc[...] = jnp.zeros_like(pooled_acc)
        cnt_acc[...] = jnp.zeros_like(cnt_acc)

    den = d0_ref[...] + d1_ref[...] + 1e-16
    h = (agg_ref[0] + agg_ref[1]) / den + bgat_ref[...]
    h = jnp.maximum(h, 0.0)

    gids = lax.broadcasted_iota(jnp.int32, (1, num_graphs), 1)
    oneh = (batch_ref[...] == gids).astype(jnp.float32)
    dims = (((0,), (0,)), ((), ()))
    pooled_acc[...] += lax.dot_general(oneh, h, dims,
                                       preferred_element_type=jnp.float32)
    ones = jnp.ones(h.shape, jnp.float32)
    cnt_acc[...] += lax.dot_general(oneh, ones, dims,
                                    preferred_element_type=jnp.float32)

    @pl.when(i == nb - 1)
    def _():
        pooled = pooled_acc[...] / jnp.maximum(cnt_acc[...], 1.0)
        out_ref[...] = jnp.dot(pooled, wlin_ref[...],
                               preferred_element_type=jnp.float32) + blin_ref[...]


def _finish(agg2, d0, d1, batch2, bgat2, W_lin, blin2, num_graphs, num_classes):
    n = agg2.shape[1]
    bm = 1000
    return pl.pallas_call(
        _finish_body,
        grid=(n // bm,),
        in_specs=[
            pl.BlockSpec((NC, bm, F_OUT), lambda i: (0, i, 0)),
            pl.BlockSpec((bm, 1), lambda i: (i, 0)),
            pl.BlockSpec((bm, 1), lambda i: (i, 0)),
            pl.BlockSpec((bm, 1), lambda i: (i, 0)),
            pl.BlockSpec((1, F_OUT), lambda i: (0, 0)),
            pl.BlockSpec((F_OUT, num_classes), lambda i: (0, 0)),
            pl.BlockSpec((1, num_classes), lambda i: (0, 0)),
        ],
        out_specs=pl.BlockSpec((num_graphs, num_classes), lambda i: (0, 0)),
        out_shape=jax.ShapeDtypeStruct((num_graphs, num_classes), jnp.float32),
        scratch_shapes=[
            pltpu.VMEM((num_graphs, F_OUT), jnp.float32),
            pltpu.VMEM((num_graphs, F_OUT), jnp.float32),
        ],
    )(agg2, d0, d1, batch2, bgat2, W_lin, blin2)


def _pieces(total):
    out, off = [], 0
    while off < total:
        sz = min(CHUNK, total - off)
        out.append((off, sz))
        off += sz
    return out


def _gather_body(xl_hbm, xr_hbm, src_hbm, dst_hbm, xlr_hbm, xrr_hbm,
                 src_v, dst_v, xl_rows, xr_rows, sem):
    c = lax.axis_index("c")
    s = lax.axis_index("s")
    wid = c * NS + s
    e_total = src_hbm.shape[0]
    per_w = e_total // (NC * NS)
    base_w = wid * per_w

    def piece(base, sz):
        pltpu.sync_copy(src_hbm.at[pl.ds(base, sz)], src_v.at[pl.ds(0, sz)])
        pltpu.sync_copy(dst_hbm.at[pl.ds(base, sz)], dst_v.at[pl.ds(0, sz)])
        a = pltpu.async_copy(xl_hbm.at[src_v.at[pl.ds(0, sz)]],
                             xl_rows.at[pl.ds(0, sz)], sem)
        b = pltpu.async_copy(xr_hbm.at[dst_v.at[pl.ds(0, sz)]],
                             xr_rows.at[pl.ds(0, sz)], sem)
        a.wait()
        b.wait()
        pltpu.sync_copy(xl_rows.at[pl.ds(0, sz)], xlr_hbm.at[pl.ds(base, sz)])
        pltpu.sync_copy(xr_rows.at[pl.ds(0, sz)], xrr_hbm.at[pl.ds(base, sz)])

    n_full = per_w // CHUNK
    tail = per_w - n_full * CHUNK

    def chunk_body(ci, cr):
        piece(base_w + ci * CHUNK, CHUNK)
        return cr

    lax.fori_loop(0, n_full, chunk_body, jnp.int32(0))
    if tail:
        piece(base_w + n_full * CHUNK, tail)


def _scatter_body(scaled_hbm, ex_hbm, dst_hbm,
                  agg_hbm, den_hbm,
                  dst_v, ex_v, rows_v, zbuf,
                  agg_sp, den_sp, sem):
    c = lax.axis_index("c")
    s = lax.axis_index("s")
    wid = c * NS + s
    e_total = dst_hbm.shape[0]
    n = agg_sp.shape[0]
    per_w = e_total // (NC * NS)
    base_w = wid * per_w

    n_t0 = (n // NS) // 8 * 8
    n_last = n - n_t0 * (NS - 1)
    nd_pad = den_sp.shape[0]
    nd_tile = nd_pad // NS

    zv = jnp.zeros((L,), jnp.float32)

    def zrow(r, cr):
        def zcol(k, ck):
            rows_v[r, pl.ds(k * L, L)] = zv
            return ck
        return lax.fori_loop(0, F_OUT // L, zcol, cr)

    lax.fori_loop(0, CHUNK_S, zrow, jnp.int32(0))

    def zflat(k, ck):
        zbuf[pl.ds(k * L, L)] = zv
        return ck

    lax.fori_loop(0, zbuf.shape[0] // L, zflat, jnp.int32(0))

    def _zero_agg_rows(r0, total):
        done = 0
        while total - done > 0:
            sz = min(CHUNK_S, total - done)
            pltpu.sync_copy(rows_v.at[pl.ds(0, sz)],
                            agg_sp.at[pl.ds(r0 + done, sz)])
            done += sz

    @pl.when(s < NS - 1)
    def _():
        _zero_agg_rows(s * n_t0, n_t0)

    @pl.when(s == NS - 1)
    def _():
        _zero_agg_rows((NS - 1) * n_t0, n_last)

    pltpu.sync_copy(zbuf.at[pl.ds(0, nd_tile)],
                    den_sp.at[pl.ds(s * nd_tile, nd_tile)])

    plsc.subcore_barrier()

    def piece(base, sz):
        pltpu.sync_copy(dst_hbm.at[pl.ds(base, sz)], dst_v.at[pl.ds(0, sz)])
        a = pltpu.async_copy(ex_hbm.at[pl.ds(base, sz)],
                             ex_v.at[pl.ds(0, sz)], sem)
        b = pltpu.async_copy(scaled_hbm.at[pl.ds(base, sz)],
                             rows_v.at[pl.ds(0, sz)], sem)
        a.wait()
        b.wait()
        pltpu.sync_copy(ex_v.at[pl.ds(0, sz)],
                        den_sp.at[dst_v.at[pl.ds(0, sz)]], add=True)
        pltpu.sync_copy(rows_v.at[pl.ds(0, sz)],
                        agg_sp.at[dst_v.at[pl.ds(0, sz)]], add=True)

    n_full = per_w // CHUNK_S
    tail = per_w - n_full * CHUNK_S

    def chunk_body(ci, cr):
        piece(base_w + ci * CHUNK_S, CHUNK_S)
        return cr

    lax.fori_loop(0, n_full, chunk_body, jnp.int32(0))
    if tail:
        piece(base_w + n_full * CHUNK_S, tail)

    plsc.subcore_barrier()

    def _drain_agg_rows(r0, total):
        done = 0
        while done < total:
            sz = min(CHUNK_S, total - done)
            pltpu.sync_copy(agg_sp.at[pl.ds(r0 + done, sz)],
                            rows_v.at[pl.ds(0, sz)])
            pltpu.sync_copy(rows_v.at[pl.ds(0, sz)],
                            agg_hbm.at[c, pl.ds(r0 + done, sz)])
            done += sz

    @pl.when(s < NS - 1)
    def _():
        _drain_agg_rows(s * n_t0, n_t0)

    @pl.when(s == NS - 1)
    def _():
        _drain_agg_rows((NS - 1) * n_t0, n_last)

    pltpu.sync_copy(den_sp.at[pl.ds(s * nd_tile, nd_tile)],
                    zbuf.at[pl.ds(0, nd_tile)])
    pltpu.sync_copy(zbuf.at[pl.ds(0, nd_tile)],
                    den_hbm.at[pl.ds(c * nd_pad + s * nd_tile, nd_tile)])


def kernel(x, edge_index, edge_attr, batch, W_l, W_r, att, W_e, b_gat,
           W_lin, b_lin):
    n = x.shape[0]
    e = edge_index.shape[1]
    num_graphs = 64
    num_classes = W_lin.shape[1]
    nd_pad = ((n + NS * 8 - 1) // (NS * 8)) * (NS * 8)

    x_l, x_r = _project(x, W_l, W_r)

    src = edge_index[0]
    dst = edge_index[1]

    mesh = plsc.VectorSubcoreMesh(core_axis_name="c", subcore_axis_name="s",
                                  num_cores=NC, num_subcores=NS)
    sc_params = pltpu.CompilerParams(needs_layout_passes=False)

    gather_fn = pl.kernel(
        _gather_body,
        out_type=[
            jax.ShapeDtypeStruct((e, F_OUT), jnp.float32),
            jax.ShapeDtypeStruct((e, F_OUT), jnp.float32),
        ],
        mesh=mesh,
        compiler_params=sc_params,
        scratch_types=[
            pltpu.VMEM((CHUNK,), jnp.int32),
            pltpu.VMEM((CHUNK,), jnp.int32),
            pltpu.VMEM((CHUNK, F_OUT), jnp.float32),
            pltpu.VMEM((CHUNK, F_OUT), jnp.float32),
            pltpu.SemaphoreType.DMA,
        ],
    )
    xlr, xrr = gather_fn(x_l, x_r, src, dst)

    ea2 = edge_attr
    we2 = W_e
    att2 = att[:, None]
    lg, bmax = _edge_logits(xlr, xrr, ea2, we2, att2)
    ex, scaled = _edge_scale(lg, xlr, bmax)

    scatter_fn = pl.kernel(
        _scatter_body,
        out_type=[
            jax.ShapeDtypeStruct((NC, n, F_OUT), jnp.float32),
            jax.ShapeDtypeStruct((NC * nd_pad,), jnp.float32),
        ],
        mesh=mesh,
        compiler_params=sc_params,
        scratch_types=[
            pltpu.VMEM((CHUNK_S,), jnp.int32),
            pltpu.VMEM((CHUNK_S,), jnp.float32),
            pltpu.VMEM((CHUNK_S, F_OUT), jnp.float32),
            pltpu.VMEM((640,), jnp.float32),
            pltpu.VMEM_SHARED((n, F_OUT), jnp.float32),
            pltpu.VMEM_SHARED((nd_pad,), jnp.float32),
            pltpu.SemaphoreType.DMA,
        ],
    )
    agg2, den2 = scatter_fn(scaled, ex[:, 0], dst)

    d0 = den2[:n, None]
    d1 = den2[nd_pad:nd_pad + n, None]
    batch2 = batch[:, None]
    bgat2 = b_gat[None, :]
    blin2 = b_lin[None, :]
    return _finish(agg2, d0, d1, batch2, bgat2, W_lin, blin2,
                   num_graphs, num_classes)
